# baseline (speedup 1.0000x reference)
import functools

import jax
import jax.numpy as jnp
from jax import lax
from jax.experimental import pallas as pl
from jax.experimental.pallas import tpu as pltpu
from jax.experimental.pallas import tpu_sc as plsc

N_NODES = 10000
D_IN = 128
D_H = 128
N_EDGES = 320000

NC, NS, L = 2, 16, 16
NW = NC * NS
BATCH = 128
EPW = 10240
NBATCH = EPW // BATCH
PAD_E = NW * EPW - N_EDGES
NP2 = 10240
NACC = 10112
RPT = NACC // NS
CH = 40
HIST = 16384

BM = 1024


def _gates_body(x_ref, h_ref, wx_ref, wh_ref, wrh_ref, bxr_ref, bxz_ref,
                bl_ref, y_ref, root_ref, z_ref):
    xb = x_ref[...]
    hb = h_ref[...]
    a = jnp.dot(xb, wx_ref[...], preferred_element_type=jnp.float32)
    b = jnp.dot(hb, wh_ref[...], preferred_element_type=jnp.float32)
    r = jax.nn.sigmoid(a[:, 0:128] + b[:, 0:128] + bxr_ref[...])
    z = jax.nn.sigmoid(a[:, 128:256] + b[:, 128:256] + bxz_ref[...])
    rh = r * hb
    c = jnp.dot(rh, wrh_ref[...], preferred_element_type=jnp.float32)
    y_ref[...] = a[:, 256:384] + c[:, 0:128]
    root_ref[...] = a[:, 384:512] + c[:, 128:256] + bl_ref[...]
    z_ref[...] = z


def _gates(x, h, wx, wh, wrh, bxr, bxz, bl):
    grid = (NP2 // BM,)
    row = lambda i: (i, 0)
    whole = lambda i: (0, 0)
    return pl.pallas_call(
        _gates_body,
        grid=grid,
        in_specs=[
            pl.BlockSpec((BM, D_IN), row),
            pl.BlockSpec((BM, D_H), row),
            pl.BlockSpec((D_IN, 512), whole),
            pl.BlockSpec((D_H, 256), whole),
            pl.BlockSpec((D_H, 256), whole),
            pl.BlockSpec((1, D_H), whole),
            pl.BlockSpec((1, D_H), whole),
            pl.BlockSpec((1, D_H), whole),
        ],
        out_specs=[
            pl.BlockSpec((BM, D_H), row),
            pl.BlockSpec((BM, D_H), row),
            pl.BlockSpec((BM, D_H), row),
        ],
        out_shape=[
            jax.ShapeDtypeStruct((NP2, D_H), jnp.float32),
            jax.ShapeDtypeStruct((NP2, D_H), jnp.float32),
            jax.ShapeDtypeStruct((NP2, D_H), jnp.float32),
        ],
    )(x, h, wx, wh, wrh, bxr, bxz, bl)


NBUF = 2


def _agg_body(y_hbm, src_hbm, dst_hbm, zacc_hbm,
              acc_out,
              src_v, dst_v, rows0, rows1, acc_sh,
              sem0, sem1):
    c = lax.axis_index("c")
    s = lax.axis_index("s")
    w = c * NS + s
    sems = (sem0, sem1)
    rows = (rows0, rows1)

    pltpu.sync_copy(zacc_hbm.at[pl.ds(s * RPT, RPT)],
                    acc_sh.at[pl.ds(s * RPT, RPT)])

    plsc.subcore_barrier()

    for chunk in range(NBATCH // CH):
        pltpu.sync_copy(src_hbm.at[w, pl.ds(chunk * CH, CH)], src_v)
        pltpu.sync_copy(dst_hbm.at[w, pl.ds(chunk * CH, CH)], dst_v)

        for b in range(NBUF):
            pltpu.async_copy(y_hbm.at[src_v.at[b]], rows[b], sems[b])

        def step(t, carry):
            for b in range(NBUF):
                j = t * NBUF + b
                pltpu.make_async_copy(y_hbm.at[src_v.at[j]],
                                      rows[b], sems[b]).wait()
                pltpu.sync_copy(rows[b], acc_sh.at[dst_v.at[j]], add=True)
                jn = j + NBUF

                @pl.when(jn < CH)
                def _():
                    pltpu.async_copy(y_hbm.at[src_v.at[jn]],
                                     rows[b], sems[b])
            return carry

        lax.fori_loop(0, CH // NBUF, step, 0)
    plsc.subcore_barrier()

    pltpu.sync_copy(acc_sh.at[pl.ds(s * RPT, RPT)],
                    acc_out.at[c, pl.ds(s * RPT, RPT)])


@functools.cache
def _agg():
    return pl.kernel(
        _agg_body,
        out_type=jax.ShapeDtypeStruct((NC, NACC, D_H), jnp.float32),
        mesh=plsc.VectorSubcoreMesh(core_axis_name="c", subcore_axis_name="s",
                                    num_cores=NC, num_subcores=NS),
        scratch_types=[
            pltpu.VMEM((CH, BATCH), jnp.int32),
            pltpu.VMEM((CH, BATCH), jnp.int32),
            pltpu.VMEM((BATCH, D_H), jnp.float32),
            pltpu.VMEM((BATCH, D_H), jnp.float32),
            pltpu.VMEM_SHARED((NACC, D_H), jnp.float32),
            pltpu.SemaphoreType.DMA,
            pltpu.SemaphoreType.DMA,
        ],
    )


def _cnt_body(dst_hbm, zhist_hbm,
              cnt_out,
              dst_v, hist_v):
    c = lax.axis_index("c")
    s = lax.axis_index("s")
    w = c * NS + s

    pltpu.sync_copy(zhist_hbm, hist_v)
    pltpu.sync_copy(dst_hbm.at[w], dst_v)

    ones16 = jnp.ones((L,), jnp.float32)

    def body(j, carry):
        for g in range(BATCH // L):
            d16 = dst_v[j, pl.ds(g * L, L)]
            plsc.addupdate_scatter(hist_v, [d16], ones16)
        return carry

    lax.fori_loop(0, NBATCH, body, 0)

    pltpu.sync_copy(hist_v, cnt_out.at[w])


@functools.cache
def _cnt():
    return pl.kernel(
        _cnt_body,
        out_type=jax.ShapeDtypeStruct((NW, HIST), jnp.float32),
        mesh=plsc.VectorSubcoreMesh(core_axis_name="c", subcore_axis_name="s",
                                    num_cores=NC, num_subcores=NS),
        scratch_types=[
            pltpu.VMEM((NBATCH, BATCH), jnp.int32),
            pltpu.VMEM((HIST,), jnp.float32),
        ],
        compiler_params=pltpu.CompilerParams(needs_layout_passes=False),
    )


def _final_body(z_ref, h_ref, root_ref, acc_ref, cnt_ref, out_ref):
    z = z_ref[...]
    cnt8 = jnp.sum(cnt_ref[...], axis=0)
    i0 = lax.broadcasted_iota(jnp.int32, (BM, 8), 0) // 128
    i1 = lax.broadcasted_iota(jnp.int32, (BM, 8), 1)
    sel = (i0 == i1).astype(jnp.float32)
    t1 = jnp.dot(sel, cnt8, preferred_element_type=jnp.float32)
    j0 = lax.broadcasted_iota(jnp.int32, (BM, 128), 0) % 128
    j1 = lax.broadcasted_iota(jnp.int32, (BM, 128), 1)
    msk = (j0 == j1).astype(jnp.float32)
    cntc = jnp.sum(t1 * msk, axis=1, keepdims=True)
    mean = (acc_ref[0] + acc_ref[1]) / jnp.maximum(cntc, 1.0)
    n = mean + root_ref[...]
    out_ref[...] = (1.0 - z) * n + z * h_ref[...]


def _final(z, h, root, acc, cnt):
    grid = (NP2 // BM,)
    row = lambda i: (i, 0)
    return pl.pallas_call(
        _final_body,
        grid=grid,
        in_specs=[
            pl.BlockSpec((BM, D_H), row),
            pl.BlockSpec((BM, D_H), row),
            pl.BlockSpec((BM, D_H), row),
            pl.BlockSpec((NC, BM, D_H), lambda i: (0, i, 0)),
            pl.BlockSpec((NW, 8, 128), lambda i: (0, i, 0)),
        ],
        out_specs=pl.BlockSpec((BM, D_H), row),
        out_shape=jax.ShapeDtypeStruct((NP2, D_H), jnp.float32),
    )(z, h, root, acc, cnt)


def kernel(x, edge_index, h_prev, Wxr, bxr, Whr, Wxz, bxz, Whz, Wl, bl, Wr):
    ei = edge_index.astype(jnp.int32)
    src = jnp.concatenate([ei[0], jnp.zeros((PAD_E,), jnp.int32)])
    pad_dst = N_NODES + jnp.arange(PAD_E, dtype=jnp.int32) % (NACC - N_NODES)
    dst = jnp.concatenate([ei[1], pad_dst])
    src = src.reshape(NW, NBATCH, BATCH)
    dst = dst.reshape(NW, NBATCH, BATCH)

    wx = jnp.concatenate([Wxr.T, Wxz.T, Wl[:, :D_IN].T, Wr[:, :D_IN].T], axis=1)
    wh = jnp.concatenate([Whr.T, Whz.T], axis=1)
    wrh = jnp.concatenate([Wl[:, D_IN:].T, Wr[:, D_IN:].T], axis=1)

    xp = jnp.pad(x, ((0, NP2 - N_NODES), (0, 0)))
    hp = jnp.pad(h_prev, ((0, NP2 - N_NODES), (0, 0)))

    y, root, z = _gates(xp, hp, wx, wh, wrh,
                        bxr[None, :], bxz[None, :], bl[None, :])

    zacc = jnp.zeros((NACC, D_H), jnp.float32)
    zhist = jnp.zeros((HIST,), jnp.float32)
    cnt = _cnt()(dst, zhist).reshape(NW, HIST // 128, 128)
    acc = _agg()(y, src, dst, zacc)
    acc = jnp.pad(acc, ((0, 0), (0, NP2 - NACC), (0, 0)))

    out = _final(z, hp, root, acc, cnt)
    return out[:N_NODES]


# speedup vs baseline: 1.0126x; 1.0126x over previous
import functools

import jax
import jax.numpy as jnp
from jax import lax
from jax.experimental import pallas as pl
from jax.experimental.pallas import tpu as pltpu
from jax.experimental.pallas import tpu_sc as plsc

N_NODES = 10000
D_IN = 128
D_H = 128
N_EDGES = 320000

NC, NS, L = 2, 16, 16
NW = NC * NS
BATCH = 128
EPW = 10240
NBATCH = EPW // BATCH
PAD_E = NW * EPW - N_EDGES
NP2 = 10240
NACC = 10112
RPT = NACC // NS
CH = 32
TOTB = NW * EPW // BATCH
NB0 = 128
NB1 = 32
HIST = 16384

BM = 1024


def _gates_body(x_ref, h_ref, wx_ref, wh_ref, wrh_ref, bxr_ref, bxz_ref,
                bl_ref, y_ref, root_ref, z_ref):
    xb = x_ref[...]
    hb = h_ref[...]
    a = jnp.dot(xb, wx_ref[...], preferred_element_type=jnp.float32)
    b = jnp.dot(hb, wh_ref[...], preferred_element_type=jnp.float32)
    r = jax.nn.sigmoid(a[:, 0:128] + b[:, 0:128] + bxr_ref[...])
    z = jax.nn.sigmoid(a[:, 128:256] + b[:, 128:256] + bxz_ref[...])
    rh = r * hb
    c = jnp.dot(rh, wrh_ref[...], preferred_element_type=jnp.float32)
    y_ref[...] = a[:, 256:384] + c[:, 0:128]
    root_ref[...] = a[:, 384:512] + c[:, 128:256] + bl_ref[...]
    z_ref[...] = z


def _gates(x, h, wx, wh, wrh, bxr, bxz, bl):
    grid = (NP2 // BM,)
    row = lambda i: (i, 0)
    whole = lambda i: (0, 0)
    return pl.pallas_call(
        _gates_body,
        grid=grid,
        in_specs=[
            pl.BlockSpec((BM, D_IN), row),
            pl.BlockSpec((BM, D_H), row),
            pl.BlockSpec((D_IN, 512), whole),
            pl.BlockSpec((D_H, 256), whole),
            pl.BlockSpec((D_H, 256), whole),
            pl.BlockSpec((1, D_H), whole),
            pl.BlockSpec((1, D_H), whole),
            pl.BlockSpec((1, D_H), whole),
        ],
        out_specs=[
            pl.BlockSpec((BM, D_H), row),
            pl.BlockSpec((BM, D_H), row),
            pl.BlockSpec((BM, D_H), row),
        ],
        out_shape=[
            jax.ShapeDtypeStruct((NP2, D_H), jnp.float32),
            jax.ShapeDtypeStruct((NP2, D_H), jnp.float32),
            jax.ShapeDtypeStruct((NP2, D_H), jnp.float32),
        ],
    )(x, h, wx, wh, wrh, bxr, bxz, bl)


NBUF = 2


def _agg_body(y_hbm, src_hbm, dst_hbm, zacc_hbm,
              acc_out,
              src_v, dst_v, rows0, rows1, acc_sh,
              sem0, sem1):
    c = lax.axis_index("c")
    s = lax.axis_index("s")
    sems = (sem0, sem1)
    rows = (rows0, rows1)

    pltpu.sync_copy(zacc_hbm.at[pl.ds(s * RPT, RPT)],
                    acc_sh.at[pl.ds(s * RPT, RPT)])

    plsc.subcore_barrier()

    base_b = jnp.where(c == 0, s * NB0, NS * NB0 + s * NB1)
    nchunks = jnp.where(c == 0, NB0 // CH, NB1 // CH)

    def chunk_body(chunk, carry):
        cb = pl.multiple_of(base_b + chunk * CH, 8)
        pltpu.sync_copy(src_hbm.at[pl.ds(cb, CH)], src_v)
        pltpu.sync_copy(dst_hbm.at[pl.ds(cb, CH)], dst_v)

        for b in range(NBUF):
            pltpu.async_copy(y_hbm.at[src_v.at[b]], rows[b], sems[b])

        def step(t, carry2):
            for b in range(NBUF):
                j = t * NBUF + b
                pltpu.make_async_copy(y_hbm.at[src_v.at[j]],
                                      rows[b], sems[b]).wait()
                pltpu.sync_copy(rows[b], acc_sh.at[dst_v.at[j]], add=True)
                jn = j + NBUF

                @pl.when(jn < CH)
                def _():
                    pltpu.async_copy(y_hbm.at[src_v.at[jn]],
                                     rows[b], sems[b])
            return carry2

        lax.fori_loop(0, CH // NBUF, step, 0)
        return carry

    lax.fori_loop(0, nchunks, chunk_body, 0)
    plsc.subcore_barrier()

    pltpu.sync_copy(acc_sh.at[pl.ds(s * RPT, RPT)],
                    acc_out.at[c, pl.ds(s * RPT, RPT)])


@functools.cache
def _agg():
    return pl.kernel(
        _agg_body,
        out_type=jax.ShapeDtypeStruct((NC, NACC, D_H), jnp.float32),
        mesh=plsc.VectorSubcoreMesh(core_axis_name="c", subcore_axis_name="s",
                                    num_cores=NC, num_subcores=NS),
        scratch_types=[
            pltpu.VMEM((CH, BATCH), jnp.int32),
            pltpu.VMEM((CH, BATCH), jnp.int32),
            pltpu.VMEM((BATCH, D_H), jnp.float32),
            pltpu.VMEM((BATCH, D_H), jnp.float32),
            pltpu.VMEM_SHARED((NACC, D_H), jnp.float32),
            pltpu.SemaphoreType.DMA,
            pltpu.SemaphoreType.DMA,
        ],
    )


def _cnt_body(dst_hbm, zhist_hbm,
              cnt_out,
              dst_v, hist_v):
    c = lax.axis_index("c")
    s = lax.axis_index("s")
    w = c * NS + s

    pltpu.sync_copy(zhist_hbm, hist_v)
    pltpu.sync_copy(dst_hbm.at[pl.ds(w * NBATCH, NBATCH)], dst_v)

    ones16 = jnp.ones((L,), jnp.float32)

    def body(j, carry):
        for g in range(BATCH // L):
            d16 = dst_v[j, pl.ds(g * L, L)]
            plsc.addupdate_scatter(hist_v, [d16], ones16)
        return carry

    lax.fori_loop(0, NBATCH, body, 0)

    pltpu.sync_copy(hist_v, cnt_out.at[w])


@functools.cache
def _cnt():
    return pl.kernel(
        _cnt_body,
        out_type=jax.ShapeDtypeStruct((NW, HIST), jnp.float32),
        mesh=plsc.VectorSubcoreMesh(core_axis_name="c", subcore_axis_name="s",
                                    num_cores=NC, num_subcores=NS),
        scratch_types=[
            pltpu.VMEM((NBATCH, BATCH), jnp.int32),
            pltpu.VMEM((HIST,), jnp.float32),
        ],
        compiler_params=pltpu.CompilerParams(needs_layout_passes=False),
    )


def _final_body(z_ref, h_ref, root_ref, acc_ref, cnt_ref, out_ref):
    z = z_ref[...]
    cnt8 = jnp.sum(cnt_ref[...], axis=0)
    i0 = lax.broadcasted_iota(jnp.int32, (BM, 8), 0) // 128
    i1 = lax.broadcasted_iota(jnp.int32, (BM, 8), 1)
    sel = (i0 == i1).astype(jnp.float32)
    t1 = jnp.dot(sel, cnt8, preferred_element_type=jnp.float32)
    j0 = lax.broadcasted_iota(jnp.int32, (BM, 128), 0) % 128
    j1 = lax.broadcasted_iota(jnp.int32, (BM, 128), 1)
    msk = (j0 == j1).astype(jnp.float32)
    cntc = jnp.sum(t1 * msk, axis=1, keepdims=True)
    mean = (acc_ref[0] + acc_ref[1]) / jnp.maximum(cntc, 1.0)
    n = mean + root_ref[...]
    out_ref[...] = (1.0 - z) * n + z * h_ref[...]


def _final(z, h, root, acc, cnt):
    grid = (NP2 // BM,)
    row = lambda i: (i, 0)
    return pl.pallas_call(
        _final_body,
        grid=grid,
        in_specs=[
            pl.BlockSpec((BM, D_H), row),
            pl.BlockSpec((BM, D_H), row),
            pl.BlockSpec((BM, D_H), row),
            pl.BlockSpec((NC, BM, D_H), lambda i: (0, i, 0)),
            pl.BlockSpec((NW, 8, 128), lambda i: (0, i, 0)),
        ],
        out_specs=pl.BlockSpec((BM, D_H), row),
        out_shape=jax.ShapeDtypeStruct((NP2, D_H), jnp.float32),
    )(z, h, root, acc, cnt)


def kernel(x, edge_index, h_prev, Wxr, bxr, Whr, Wxz, bxz, Whz, Wl, bl, Wr):
    ei = edge_index.astype(jnp.int32)
    src = jnp.concatenate([ei[0], jnp.zeros((PAD_E,), jnp.int32)])
    pad_dst = N_NODES + jnp.arange(PAD_E, dtype=jnp.int32) % (NACC - N_NODES)
    dst = jnp.concatenate([ei[1], pad_dst])
    src = src.reshape(TOTB, BATCH)
    dst = dst.reshape(TOTB, BATCH)

    wx = jnp.concatenate([Wxr.T, Wxz.T, Wl[:, :D_IN].T, Wr[:, :D_IN].T], axis=1)
    wh = jnp.concatenate([Whr.T, Whz.T], axis=1)
    wrh = jnp.concatenate([Wl[:, D_IN:].T, Wr[:, D_IN:].T], axis=1)

    xp = jnp.pad(x, ((0, NP2 - N_NODES), (0, 0)))
    hp = jnp.pad(h_prev, ((0, NP2 - N_NODES), (0, 0)))

    y, root, z = _gates(xp, hp, wx, wh, wrh,
                        bxr[None, :], bxz[None, :], bl[None, :])

    zacc = jnp.zeros((NACC, D_H), jnp.float32)
    zhist = jnp.zeros((HIST,), jnp.float32)
    cnt = _cnt()(dst, zhist).reshape(NW, HIST // 128, 128)
    acc = _agg()(y, src, dst, zacc)
    acc = jnp.pad(acc, ((0, 0), (0, NP2 - NACC), (0, 0)))

    out = _final(z, hp, root, acc, cnt)
    return out[:N_NODES]
